# baseline (speedup 1.0000x reference)
import functools

import jax
import jax.numpy as jnp
from jax import lax
from jax.experimental import pallas as pl
from jax.experimental.pallas import tpu as pltpu
from jax.experimental.pallas import tpu_sc as plsc

N = 10000
E = 160000
H = 256
DE = 16
L = 3

NC, NS, LANES = 2, 16, 16
N_PAD = 10240
E_PAD = 163840
EPT = E_PAD // NS
NB = EPT // 128
RPT = N_PAD // NS
CH_W = 128
NCH = 4
NC_SC = 2
R_ROWS = N_PAD // 2
R_BLKS = R_ROWS // 128
ACC_ROWS = R_ROWS + 128

_HI = jax.lax.Precision.HIGHEST


def _dot(a, b):
    return jax.lax.dot_general(a, b, (((1,), (0,)), ((), ())),
                               precision=_HI, preferred_element_type=jnp.float32)


def _ln_tc(x, g, b):
    m = jnp.mean(x, axis=-1, keepdims=True)
    v = jnp.mean((x - m) * (x - m), axis=-1, keepdims=True)
    return (x - m) * jax.lax.rsqrt(v + 1e-5) * g + b


def _edge_pre_body(ea_ref, epw_ref, epb_ref, w1r_ref, b1_ref, out_ref):
    ea = ea_ref[...]
    for l in range(L):
        wc = _dot(epw_ref[...], w1r_ref[l])
        beff = _dot(epb_ref[...], w1r_ref[l]) + b1_ref[l][None, :]
        y = _dot(ea, wc) + beff
        for c in range(NCH):
            out_ref[l * NCH + c] = y[:, c * CH_W:(c + 1) * CH_W]


def _edge_pre(ea_pad, ep_W, ep_b, w1r, msg_b1):
    BE = 2048
    nb = E_PAD // BE
    return pl.pallas_call(
        _edge_pre_body,
        grid=(nb,),
        in_specs=[
            pl.BlockSpec((BE, DE), lambda i: (i, 0)),
            pl.BlockSpec((DE, H), lambda i: (0, 0)),
            pl.BlockSpec((1, H), lambda i: (0, 0)),
            pl.BlockSpec((L, H, 2 * H), lambda i: (0, 0, 0)),
            pl.BlockSpec((L, 2 * H), lambda i: (0, 0)),
        ],
        out_specs=pl.BlockSpec((L * NCH, BE, CH_W), lambda i: (0, i, 0)),
        out_shape=jax.ShapeDtypeStruct((L * NCH, E_PAD, CH_W), jnp.float32),
    )(ea_pad, ep_W, ep_b.reshape(1, H), w1r, msg_b1)


def _tables_body(h_ref, w1s_ref, w1d_ref, os_ref, od_ref):
    h = h_ref[...]
    os_ref[...] = _dot(h, w1s_ref[...])
    od_ref[...] = _dot(h, w1d_ref[...])


def _tables(h_pad, w1s, w1d):
    BN = 1024
    nb = N_PAD // BN
    return pl.pallas_call(
        _tables_body,
        grid=(nb,),
        in_specs=[
            pl.BlockSpec((BN, H), lambda i: (i, 0)),
            pl.BlockSpec((H, 2 * H), lambda i: (0, 0)),
            pl.BlockSpec((H, 2 * H), lambda i: (0, 0)),
        ],
        out_specs=[
            pl.BlockSpec((BN, 2 * H), lambda i: (i, 0)),
            pl.BlockSpec((BN, 2 * H), lambda i: (i, 0)),
        ],
        out_shape=[
            jax.ShapeDtypeStruct((N_PAD, 2 * H), jnp.float32),
            jax.ShapeDtypeStruct((N_PAD, 2 * H), jnp.float32),
        ],
    )(h_pad, w1s, w1d)


def _node_common(h_ref, a_refs, deg_ref, w2_ref, b2_ref, ng_ref, nb_ref,
                 f1_ref, fb1_ref, f2_ref, fb2_ref, fg_ref, fb_ref):
    agg = _dot(a_refs[0][...], w2_ref[0:CH_W, :])
    for c in range(1, NCH):
        agg = agg + _dot(a_refs[c][...], w2_ref[c * CH_W:(c + 1) * CH_W, :])
    deg = deg_ref[:, 0:1]
    inv = 1.0 / jnp.maximum(deg, 1.0)
    agg = agg * inv + b2_ref[...] * (deg * inv)
    h1 = _ln_tc(h_ref[...] + agg, ng_ref[...], nb_ref[...])
    t = _dot(h1, f1_ref[...]) + fb1_ref[...]
    f = _dot(t * jax.nn.sigmoid(t), f2_ref[...]) + fb2_ref[...]
    h2 = _ln_tc(h1 + f, fg_ref[...], fb_ref[...])
    return h2


def _node_mid_body(h_ref, *rest):
    a_refs = rest[:NCH]
    (deg_ref, w2_ref, b2_ref, ng_ref, nb_ref, f1_ref, fb1_ref, f2_ref,
     fb2_ref, fg_ref, fb_ref, w1s_ref, w1d_ref,
     oh_ref, os_ref, od_ref) = rest[NCH:]
    h2 = _node_common(h_ref, a_refs, deg_ref, w2_ref, b2_ref,
                      ng_ref, nb_ref, f1_ref, fb1_ref, f2_ref, fb2_ref,
                      fg_ref, fb_ref)
    oh_ref[...] = h2
    os_ref[...] = _dot(h2, w1s_ref[...])
    od_ref[...] = _dot(h2, w1d_ref[...])


def _node_last_body(h_ref, *rest):
    a_refs = rest[:NCH]
    (deg_ref, w2_ref, b2_ref, ng_ref, nb_ref, f1_ref, fb1_ref, f2_ref,
     fb2_ref, fg_ref, fb_ref, ow_ref, ob_ref, o_ref) = rest[NCH:]
    h2 = _node_common(h_ref, a_refs, deg_ref, w2_ref, b2_ref,
                      ng_ref, nb_ref, f1_ref, fb1_ref, f2_ref, fb2_ref,
                      fg_ref, fb_ref)
    o_ref[...] = _dot(h2, ow_ref[...]) + ob_ref[...]


def _node_specs(BN):
    vec = lambda: pl.BlockSpec((1, H), lambda i: (0, 0))
    vec2 = lambda: pl.BlockSpec((1, 2 * H), lambda i: (0, 0))
    return [
        pl.BlockSpec((BN, H), lambda i: (i, 0)),
    ] + [
        pl.BlockSpec((BN, CH_W), lambda i: (i, 0))
        for _ in range(NCH)
    ] + [
        pl.BlockSpec((BN, LANES), lambda i: (i, 0)),
        pl.BlockSpec((2 * H, H), lambda i: (0, 0)),
        vec(),
        vec(), vec(),
        pl.BlockSpec((H, 2 * H), lambda i: (0, 0)),
        vec2(),
        pl.BlockSpec((2 * H, H), lambda i: (0, 0)),
        vec(),
        vec(), vec(),
    ]


def _node_mid(h_pad, accs, degcnt, wts, w1s_n, w1d_n):
    BN = 1024
    nb = N_PAD // BN
    specs = _node_specs(BN) + [
        pl.BlockSpec((H, 2 * H), lambda i: (0, 0)),
        pl.BlockSpec((H, 2 * H), lambda i: (0, 0)),
    ]
    return pl.pallas_call(
        _node_mid_body,
        grid=(nb,),
        in_specs=specs,
        out_specs=[
            pl.BlockSpec((BN, H), lambda i: (i, 0)),
            pl.BlockSpec((BN, 2 * H), lambda i: (i, 0)),
            pl.BlockSpec((BN, 2 * H), lambda i: (i, 0)),
        ],
        out_shape=[
            jax.ShapeDtypeStruct((N_PAD, H), jnp.float32),
            jax.ShapeDtypeStruct((N_PAD, 2 * H), jnp.float32),
            jax.ShapeDtypeStruct((N_PAD, 2 * H), jnp.float32),
        ],
    )(h_pad, *accs, degcnt, *wts, w1s_n, w1d_n)


def _node_last(h_pad, accs, degcnt, wts, out_W, out_b):
    BN = 1024
    nb = N_PAD // BN
    specs = _node_specs(BN) + [
        pl.BlockSpec((H, H), lambda i: (0, 0)),
        pl.BlockSpec((1, H), lambda i: (0, 0)),
    ]
    return pl.pallas_call(
        _node_last_body,
        grid=(nb,),
        in_specs=specs,
        out_specs=pl.BlockSpec((BN, H), lambda i: (i, 0)),
        out_shape=jax.ShapeDtypeStruct((N_PAD, H), jnp.float32),
    )(h_pad, *accs, degcnt, *wts, out_W, out_b.reshape(1, H))


_MESH = plsc.VectorSubcoreMesh(core_axis_name="c", subcore_axis_name="s",
                               num_cores=NC_SC, num_subcores=NS)


def _make_sc_layer(layer):
    @functools.partial(
        pl.kernel,
        out_type=jax.ShapeDtypeStruct((NCH * N_PAD, CH_W), jnp.float32),
        mesh=_MESH,
        scratch_types=[
            pltpu.VMEM((NB, 128), jnp.int32),
            pltpu.VMEM((NB, 128), jnp.int32),
            pltpu.VMEM((128,), jnp.int32),
            pltpu.VMEM((128,), jnp.int32),
            pltpu.VMEM((128,), jnp.int32),
            pltpu.VMEM((128, CH_W), jnp.float32),
            pltpu.VMEM((128, CH_W), jnp.float32),
            pltpu.VMEM((128, CH_W), jnp.float32),
            pltpu.VMEM((128, CH_W), jnp.float32),
            pltpu.VMEM_SHARED((ACC_ROWS, CH_W), jnp.float32),
            pltpu.SemaphoreType.DMA,
            pltpu.SemaphoreType.DMA,
        ],
    )
    def sc_layer(hs_hbm, hd_hbm, pre_hbm, src_hbm, dst_hbm, zeros_hbm,
                 out_hbm, sidx, didx, sgi, dgi, ssi, gs, gd, pv, sb, acc,
                 sem1, sem2):
        c = lax.axis_index("c")
        s = lax.axis_index("s")
        rbase = c * R_ROWS

        pltpu.sync_copy(src_hbm.at[pl.ds(s * NB, NB)], sidx)
        pltpu.sync_copy(dst_hbm.at[pl.ds(s * NB, NB)], didx)

        nzb = ACC_ROWS // 128
        per_tile = (nzb + NS - 1) // NS
        for p in range(NCH):
            ch = p
            off = ch * N_PAD
            pltpu.sync_copy(zeros_hbm, pv)
            for j in range(per_tile):
                blk = j * NS + s

                @pl.when(blk < nzb)
                def _():
                    pltpu.sync_copy(pv, acc.at[pl.ds(blk * 128, 128)])
            plsc.subcore_barrier()

            def batch(b, carry):
                def mkidx(k, carry2):
                    sl = pl.ds(k * LANES, LANES)
                    sv = sidx[b, sl]
                    dv = didx[b, sl]
                    sgi[sl] = sv + off
                    dgi[sl] = dv + off
                    rel = dv - rbase
                    inr = (rel >= 0) & (rel < R_ROWS)
                    trash = R_ROWS + (dv & 127)
                    ssi[sl] = jnp.where(inr, rel, trash)
                    return carry2
                lax.fori_loop(0, 128 // LANES, mkidx, 0)
                cps = pltpu.async_copy(hs_hbm.at[sgi], gs, sem1)
                cpd = pltpu.async_copy(hd_hbm.at[dgi], gd, sem2)
                pre_row = (layer * NCH * E_PAD + s * EPT + b * 128
                           + ch * E_PAD)
                pltpu.sync_copy(pre_hbm.at[pl.ds(pre_row, 128)], pv)
                cps.wait()
                cpd.wait()

                def rowloop(r, carry2):
                    for k in range(CH_W // LANES):
                        sl = pl.ds(k * LANES, LANES)
                        x = gs[r, sl] + gd[r, sl] + pv[r, sl]
                        sb[r, sl] = x / (1.0 + jnp.exp(-x))
                    return carry2
                lax.fori_loop(0, 128, rowloop, 0)
                pltpu.sync_copy(sb, acc.at[ssi], add=True)
                return carry
            lax.fori_loop(0, NB, batch, 0)
            plsc.subcore_barrier()

            for j in range(per_tile):
                blk = j * NS + s

                @pl.when(blk < R_BLKS)
                def _():
                    pltpu.sync_copy(acc.at[pl.ds(blk * 128, 128)], gs)
                    pltpu.sync_copy(
                        gs, out_hbm.at[pl.ds(off + rbase + blk * 128, 128)])
            plsc.subcore_barrier()

    return sc_layer


@functools.partial(
    pl.kernel,
    out_type=jax.ShapeDtypeStruct((N_PAD, CH_W), jnp.float32),
    mesh=_MESH,
    scratch_types=[
        pltpu.VMEM((NB, 128), jnp.int32),
        pltpu.VMEM((128,), jnp.int32),
        pltpu.VMEM((128, CH_W), jnp.float32),
        pltpu.VMEM((128, CH_W), jnp.float32),
        pltpu.VMEM_SHARED((ACC_ROWS, CH_W), jnp.float32),
    ],
)
def _sc_deg(dst_hbm, ones_hbm, zeros_hbm, out_hbm, didx, ssi, ov, buf, dacc):
    c = lax.axis_index("c")
    s = lax.axis_index("s")
    rbase = c * R_ROWS

    pltpu.sync_copy(dst_hbm.at[pl.ds(s * NB, NB)], didx)
    pltpu.sync_copy(ones_hbm, ov)
    pltpu.sync_copy(zeros_hbm, buf)
    nzb = ACC_ROWS // 128
    for j in range((nzb + NS - 1) // NS):
        blk = j * NS + s

        @pl.when(blk < nzb)
        def _():
            pltpu.sync_copy(buf, dacc.at[pl.ds(blk * 128, 128)])
    plsc.subcore_barrier()

    def batch(b, carry):
        def mkidx(k, carry2):
            sl = pl.ds(k * LANES, LANES)
            dv = didx[b, sl]
            rel = dv - rbase
            inr = (rel >= 0) & (rel < R_ROWS)
            ssi[sl] = jnp.where(inr, rel, R_ROWS + (dv & 127))
            return carry2
        lax.fori_loop(0, 128 // LANES, mkidx, 0)
        pltpu.sync_copy(ov, dacc.at[ssi], add=True)
        return carry
    lax.fori_loop(0, NB, batch, 0)
    plsc.subcore_barrier()

    for j in range((nzb + NS - 1) // NS):
        blk = j * NS + s

        @pl.when(blk < R_BLKS)
        def _():
            pltpu.sync_copy(dacc.at[pl.ds(blk * 128, 128)], buf)
            pltpu.sync_copy(buf, out_hbm.at[pl.ds(rbase + blk * 128, 128)])


_SC_LAYERS = [_make_sc_layer(l) for l in range(L)]


def kernel(h, edge_index, edge_attr, ep_W, ep_b, msg_W1, msg_b1, msg_W2,
           msg_b2, norm_g, norm_b, ffn_W1, ffn_b1, ffn_W2, ffn_b2, fnorm_g,
           fnorm_b, out_W, out_b):
    f32 = jnp.float32
    h0 = h[0]
    src = edge_index[0, :, 0]
    dst = edge_index[0, :, 1]
    ea = edge_attr[0]

    npad_e = E_PAD - E
    pad_ids = (N + (jnp.arange(npad_e, dtype=jnp.int32) % LANES)).astype(jnp.int32)
    src_pad = jnp.concatenate([src.astype(jnp.int32), pad_ids])
    dst_pad = jnp.concatenate([dst.astype(jnp.int32), pad_ids])
    src2d = src_pad.reshape(E_PAD // 128, 128)
    dst2d = dst_pad.reshape(E_PAD // 128, 128)
    ea_pad = jnp.concatenate([ea, jnp.zeros((npad_e, DE), f32)], axis=0)
    h_pad = jnp.concatenate([h0, jnp.zeros((N_PAD - N, H), f32)], axis=0)

    zeros_ch = jnp.zeros((128, CH_W), f32)
    ones_ch = jnp.ones((128, CH_W), f32)

    w1s = msg_W1[:, :H, :]
    w1d = msg_W1[:, H:2 * H, :]
    w1r = msg_W1[:, 2 * H:, :]

    pre = _edge_pre(ea_pad, ep_W, ep_b, w1r, msg_b1)
    pre_flat = pre.reshape(L * NCH * E_PAD, CH_W)

    degcnt = _sc_deg(dst2d, ones_ch, zeros_ch)[:, :LANES]

    hws, hwd = _tables(h_pad, w1s[0], w1d[0])

    h_cur = h_pad
    for l in range(L):
        hs_all = jnp.concatenate(
            [hws[:, c * CH_W:(c + 1) * CH_W] for c in range(NCH)], axis=0)
        hd_all = jnp.concatenate(
            [hwd[:, c * CH_W:(c + 1) * CH_W] for c in range(NCH)], axis=0)
        acc = _SC_LAYERS[l](hs_all, hd_all, pre_flat, src2d, dst2d, zeros_ch)
        accs = [acc[c * N_PAD:(c + 1) * N_PAD] for c in range(NCH)]
        wts = (msg_W2[l], msg_b2[l].reshape(1, H), norm_g[l].reshape(1, H),
               norm_b[l].reshape(1, H), ffn_W1[l], ffn_b1[l].reshape(1, 2 * H),
               ffn_W2[l], ffn_b2[l].reshape(1, H), fnorm_g[l].reshape(1, H),
               fnorm_b[l].reshape(1, H))
        if l < L - 1:
            h_cur, hws, hwd = _node_mid(h_cur, accs, degcnt, wts,
                                        w1s[l + 1], w1d[l + 1])
        else:
            res = _node_last(h_cur, accs, degcnt, wts, out_W, out_b)
    return res[:N][None]
